# baseline (speedup 1.0000x reference)
import functools

import jax
import jax.numpy as jnp
import numpy as np
from jax.experimental import pallas as pl
from jax.sharding import Mesh, PartitionSpec as P

try:
    from jax.experimental.shard_map import shard_map as _shard_map
except ImportError:
    _shard_map = jax.shard_map


def _prep_kernel(x_ref, xb_ref, xtb_ref):
    xb = x_ref[...].astype(jnp.bfloat16)
    xb_ref[...] = xb
    xtb_ref[...] = xb.T


def _masked_dots(e_num, xb, lhs_bf16, lhs_sum):
    parts = []
    for e in range(e_num - 1):
        m = jnp.where(xb == e, jnp.bfloat16(1), jnp.bfloat16(0))
        parts.append(jnp.dot(lhs_bf16, m, preferred_element_type=jnp.float32))
    total = parts[0]
    for p in parts[1:]:
        total = total + p
    last = lhs_sum - total
    return jnp.concatenate(parts + [last], axis=0)


def _pass_a_kernel(e_num, xt_ref, ut_ref, at_ref, w2_ref, out_ref):
    u = ut_ref[...].astype(jnp.bfloat16)
    usum = jnp.sum(u.astype(jnp.float32), axis=1, keepdims=True)
    s = _masked_dots(e_num, xt_ref[...], u, usum)
    m_a = jnp.dot(w2_ref[...], s, preferred_element_type=jnp.float32)
    out_ref[...] = at_ref[...] + m_a


def _pass_b_kernel(e_num, x_ref, at_ref, w1_ref, out_ref):
    a = at_ref[...].astype(jnp.bfloat16)
    asum = jnp.sum(a.astype(jnp.float32), axis=1, keepdims=True)
    st = _masked_dots(e_num, x_ref[...], a, asum)
    out_ref[...] = jnp.dot(w1_ref[...], st, preferred_element_type=jnp.float32)


def _softmax_kernel(ut_ref, mt_ref, out_ref):
    z = ut_ref[...] + mt_ref[...]
    z = z - jnp.max(z, axis=0, keepdims=True)
    p = jnp.exp(z)
    out_ref[...] = p / jnp.sum(p, axis=0, keepdims=True)


def _pass_a(xtb, ut, at, w2r, block):
    a_num, w_num = at.shape
    e_num, t_num = ut.shape
    return pl.pallas_call(
        functools.partial(_pass_a_kernel, e_num),
        grid=(w_num // block,),
        in_specs=[
            pl.BlockSpec((t_num, block), lambda j: (0, j)),
            pl.BlockSpec((e_num, t_num), lambda j: (0, 0)),
            pl.BlockSpec((a_num, block), lambda j: (0, j)),
            pl.BlockSpec(w2r.shape, lambda j: (0, 0)),
        ],
        out_specs=pl.BlockSpec((a_num, block), lambda j: (0, j)),
        out_shape=jax.ShapeDtypeStruct((a_num, w_num), jnp.float32),
    )(xtb, ut, at, w2r)


def _pass_b(xb, at, w1r, t_num, block):
    a_num, w_num = at.shape
    e_num = w1r.shape[0]
    return pl.pallas_call(
        functools.partial(_pass_b_kernel, e_num),
        grid=(t_num // block,),
        in_specs=[
            pl.BlockSpec((w_num, block), lambda k: (0, k)),
            pl.BlockSpec((a_num, w_num), lambda k: (0, 0)),
            pl.BlockSpec(w1r.shape, lambda k: (0, 0)),
        ],
        out_specs=pl.BlockSpec((e_num, block), lambda k: (0, k)),
        out_shape=jax.ShapeDtypeStruct((e_num, t_num), jnp.float32),
    )(xb, at, w1r)


def _softmax_update(ut, mt):
    e_num, t_num = ut.shape
    return pl.pallas_call(
        _softmax_kernel,
        grid=(1,),
        in_specs=[
            pl.BlockSpec((e_num, t_num), lambda i: (0, 0)),
            pl.BlockSpec((e_num, t_num), lambda i: (0, 0)),
        ],
        out_specs=pl.BlockSpec((e_num, t_num), lambda i: (0, 0)),
        out_shape=jax.ShapeDtypeStruct((e_num, t_num), jnp.float32),
    )(ut, mt)


def _prep(x_int, prep_block):
    w_loc, t_num = x_int.shape
    return pl.pallas_call(
        _prep_kernel,
        grid=(w_loc // prep_block,),
        in_specs=[pl.BlockSpec((prep_block, t_num), lambda i: (i, 0))],
        out_specs=[
            pl.BlockSpec((prep_block, t_num), lambda i: (i, 0)),
            pl.BlockSpec((t_num, prep_block), lambda i: (0, i)),
        ],
        out_shape=[
            jax.ShapeDtypeStruct((w_loc, t_num), jnp.bfloat16),
            jax.ShapeDtypeStruct((t_num, w_loc), jnp.bfloat16),
        ],
    )(x_int)


def _forward_sharded(update_step, block, x_loc, at_loc, ut, w2r, w1r):
    xb_loc, xtb_loc = _prep(x_loc, 256)
    for _ in range(update_step):
        at_loc = _pass_a(xtb_loc, ut, at_loc, w2r, block)
        mt_part = _pass_b(xb_loc, at_loc, w1r, ut.shape[1], block)
        mt = jax.lax.psum(mt_part, "d")
        ut = _softmax_update(ut, mt)
    return at_loc, ut


def kernel(first_a, first_t, padding_a, padding_t, Awij, Awij2, inputs):
    e_num, a_num, _ = Awij.shape
    w_num, t_num = inputs.shape
    update_step = 2
    block = 256

    at = first_a.T
    ut = first_t.T
    w2r = jnp.transpose(Awij2, (2, 0, 1)).reshape(a_num, e_num * e_num)
    w1r = jnp.transpose(Awij, (2, 0, 1)).reshape(e_num, e_num * a_num)

    devs = jax.devices()
    n_shards = 2 if len(devs) >= 2 and w_num % (2 * block) == 0 else 1
    mesh = Mesh(np.array(devs[:n_shards]), ("d",))
    body = functools.partial(_forward_sharded, update_step, block)
    in_specs = (P("d", None), P(None, "d"), P(None, None), P(None, None),
                P(None, None))
    out_specs = (P(None, "d"), P(None, None))
    try:
        fwd = _shard_map(body, mesh=mesh, in_specs=in_specs,
                         out_specs=out_specs, check_rep=False)
    except TypeError:
        fwd = _shard_map(body, mesh=mesh, in_specs=in_specs,
                         out_specs=out_specs, check_vma=False)
    at, ut = fwd(inputs, at, ut, w2r, w1r)

    top = jnp.concatenate([at.T, padding_a], axis=1)
    bot = jnp.concatenate([ut.T, padding_t], axis=1)
    return jnp.concatenate([top, bot], axis=0)


# speedup vs baseline: 2.1862x; 2.1862x over previous
import functools

import jax
import jax.numpy as jnp
from jax.experimental import pallas as pl


def _prep_kernel(x_ref, xb_ref, xtb_ref):
    xb = x_ref[...].astype(jnp.bfloat16)
    xb_ref[...] = xb
    xtb_ref[...] = xb.T


def _masked_dots(e_num, xb, lhs_bf16, lhs_sum):
    parts = []
    for e in range(e_num - 1):
        m = jnp.where(xb == e, jnp.bfloat16(1), jnp.bfloat16(0))
        parts.append(jnp.dot(lhs_bf16, m, preferred_element_type=jnp.float32))
    total = parts[0]
    for p in parts[1:]:
        total = total + p
    last = lhs_sum - total
    return jnp.concatenate(parts + [last], axis=0)


def _pass_a_kernel(e_num, xt_ref, ut_ref, at_ref, w2_ref, out_ref):
    u = ut_ref[...].astype(jnp.bfloat16)
    usum = jnp.sum(u.astype(jnp.float32), axis=1, keepdims=True)
    s = _masked_dots(e_num, xt_ref[...], u, usum)
    m_a = jnp.dot(w2_ref[...], s, preferred_element_type=jnp.float32)
    out_ref[...] = at_ref[...] + m_a


def _pass_b_kernel(e_num, x_ref, at_ref, ut_ref, w1_ref, out_ref):
    a = at_ref[...].astype(jnp.bfloat16)
    asum = jnp.sum(a.astype(jnp.float32), axis=1, keepdims=True)
    st = _masked_dots(e_num, x_ref[...], a, asum)
    m_t = jnp.dot(w1_ref[...], st, preferred_element_type=jnp.float32)
    z = ut_ref[...] + m_t
    z = z - jnp.max(z, axis=0, keepdims=True)
    p = jnp.exp(z)
    out_ref[...] = p / jnp.sum(p, axis=0, keepdims=True)


def _pass_a(xtb, ut, at, w2r, block):
    a_num, w_num = at.shape
    e_num, t_num = ut.shape
    return pl.pallas_call(
        functools.partial(_pass_a_kernel, e_num),
        grid=(w_num // block,),
        in_specs=[
            pl.BlockSpec((t_num, block), lambda j: (0, j)),
            pl.BlockSpec((e_num, t_num), lambda j: (0, 0)),
            pl.BlockSpec((a_num, block), lambda j: (0, j)),
            pl.BlockSpec(w2r.shape, lambda j: (0, 0)),
        ],
        out_specs=pl.BlockSpec((a_num, block), lambda j: (0, j)),
        out_shape=jax.ShapeDtypeStruct((a_num, w_num), jnp.float32),
    )(xtb, ut, at, w2r)


def _pass_b(xb, at, ut, w1r, block):
    a_num, w_num = at.shape
    e_num, t_num = ut.shape
    return pl.pallas_call(
        functools.partial(_pass_b_kernel, e_num),
        grid=(t_num // block,),
        in_specs=[
            pl.BlockSpec((w_num, block), lambda k: (0, k)),
            pl.BlockSpec((a_num, w_num), lambda k: (0, 0)),
            pl.BlockSpec((e_num, block), lambda k: (0, k)),
            pl.BlockSpec(w1r.shape, lambda k: (0, 0)),
        ],
        out_specs=pl.BlockSpec((e_num, block), lambda k: (0, k)),
        out_shape=jax.ShapeDtypeStruct((e_num, t_num), jnp.float32),
    )(xb, at, ut, w1r)


def kernel(first_a, first_t, padding_a, padding_t, Awij, Awij2, inputs):
    e_num, a_num, _ = Awij.shape
    update_step = 2
    block = 512

    w_num, t_num = inputs.shape
    prep_block = 256
    xb, xtb = pl.pallas_call(
        _prep_kernel,
        grid=(w_num // prep_block,),
        in_specs=[pl.BlockSpec((prep_block, t_num), lambda i: (i, 0))],
        out_specs=[
            pl.BlockSpec((prep_block, t_num), lambda i: (i, 0)),
            pl.BlockSpec((t_num, prep_block), lambda i: (0, i)),
        ],
        out_shape=[
            jax.ShapeDtypeStruct((w_num, t_num), jnp.bfloat16),
            jax.ShapeDtypeStruct((t_num, w_num), jnp.bfloat16),
        ],
    )(inputs)
    at = first_a.T
    ut = first_t.T
    w2r = jnp.transpose(Awij2, (2, 0, 1)).reshape(a_num, e_num * e_num)
    w1r = jnp.transpose(Awij, (2, 0, 1)).reshape(e_num, e_num * a_num)

    for _ in range(update_step):
        at = _pass_a(xtb, ut, at, w2r, block)
        ut = _pass_b(xb, at, ut, w1r, block)

    top = jnp.concatenate([at.T, padding_a], axis=1)
    bot = jnp.concatenate([ut.T, padding_t], axis=1)
    return jnp.concatenate([top, bot], axis=0)


# speedup vs baseline: 2.2454x; 1.0271x over previous
import functools

import jax
import jax.numpy as jnp
from jax.experimental import pallas as pl
from jax.experimental.pallas import tpu as pltpu


def _prep_kernel(x_ref, xb_ref, xtb_ref):
    xb = x_ref[...].astype(jnp.bfloat16)
    xb_ref[...] = xb
    xtb_ref[...] = xb.T


def _masked_dots(e_num, xb, lhs_bf16, lhs_sum):
    parts = []
    for e in range(e_num - 1):
        m = jnp.where(xb == e, jnp.bfloat16(1), jnp.bfloat16(0))
        parts.append(jnp.dot(lhs_bf16, m, preferred_element_type=jnp.float32))
    total = parts[0]
    for p in parts[1:]:
        total = total + p
    last = lhs_sum - total
    return jnp.concatenate(parts + [last], axis=0)


def _fused_kernel(nblk, block, e_num, xtb_ref, xb_ref, at0_ref, ut0_ref,
                  w2_ref, w1_ref, at_out_ref, ut_out_ref, at_s, ut_s):
    p = pl.program_id(0)
    b = pl.program_id(1)

    @pl.when((p == 0) & (b == 0))
    def _init():
        at_s[...] = at0_ref[...]
        ut_s[...] = ut0_ref[...]

    @pl.when((p == 0) | (p == 2))
    def _worker_pass():
        u = ut_s[...].astype(jnp.bfloat16)
        usum = jnp.sum(u.astype(jnp.float32), axis=1, keepdims=True)
        s = _masked_dots(e_num, xtb_ref[...], u, usum)
        m_a = jnp.dot(w2_ref[...], s, preferred_element_type=jnp.float32)
        sl = pl.ds(b * block, block)
        at_s[:, sl] = at_s[:, sl] + m_a

    @pl.when((p == 1) | (p == 3))
    def _task_pass():
        a = at_s[...].astype(jnp.bfloat16)
        asum = jnp.sum(a.astype(jnp.float32), axis=1, keepdims=True)
        st = _masked_dots(e_num, xb_ref[...], a, asum)
        m_t = jnp.dot(w1_ref[...], st, preferred_element_type=jnp.float32)
        sl = pl.ds(b * block, block)
        z = ut_s[:, sl] + m_t
        z = z - jnp.max(z, axis=0, keepdims=True)
        q = jnp.exp(z)
        unew = q / jnp.sum(q, axis=0, keepdims=True)
        ut_s[:, sl] = unew

        @pl.when(p == 3)
        def _emit_t():
            ut_out_ref[...] = unew

    @pl.when((p == 3) & (b == nblk - 1))
    def _emit_a():
        at_out_ref[...] = at_s[...]


def kernel(first_a, first_t, padding_a, padding_t, Awij, Awij2, inputs):
    e_num, a_num, _ = Awij.shape
    w_num, t_num = inputs.shape
    block = 512
    nblk = w_num // block

    prep_block = 256
    xb, xtb = pl.pallas_call(
        _prep_kernel,
        grid=(w_num // prep_block,),
        in_specs=[pl.BlockSpec((prep_block, t_num), lambda i: (i, 0))],
        out_specs=[
            pl.BlockSpec((prep_block, t_num), lambda i: (i, 0)),
            pl.BlockSpec((t_num, prep_block), lambda i: (0, i)),
        ],
        out_shape=[
            jax.ShapeDtypeStruct((w_num, t_num), jnp.bfloat16),
            jax.ShapeDtypeStruct((t_num, w_num), jnp.bfloat16),
        ],
    )(inputs)
    at = first_a.T
    ut = first_t.T
    w2r = jnp.transpose(Awij2, (2, 0, 1)).reshape(a_num, e_num * e_num)
    w1r = jnp.transpose(Awij, (2, 0, 1)).reshape(e_num, e_num * a_num)

    def _a_phase(p, b):
        return ((p == 0) | (p == 2)).astype(jnp.int32)

    at_new, ut_new = pl.pallas_call(
        functools.partial(_fused_kernel, nblk, block, e_num),
        grid=(4, nblk),
        in_specs=[
            pl.BlockSpec((t_num, block), lambda p, b: (0, b * _a_phase(p, b))),
            pl.BlockSpec((w_num, block),
                         lambda p, b: (0, b * (1 - _a_phase(p, b)))),
            pl.BlockSpec((a_num, w_num), lambda p, b: (0, 0)),
            pl.BlockSpec((e_num, t_num), lambda p, b: (0, 0)),
            pl.BlockSpec(w2r.shape, lambda p, b: (0, 0)),
            pl.BlockSpec(w1r.shape, lambda p, b: (0, 0)),
        ],
        out_specs=[
            pl.BlockSpec((a_num, w_num), lambda p, b: (0, 0)),
            pl.BlockSpec((e_num, block),
                         lambda p, b: (0, b * (p == 3).astype(jnp.int32))),
        ],
        out_shape=[
            jax.ShapeDtypeStruct((a_num, w_num), jnp.float32),
            jax.ShapeDtypeStruct((e_num, t_num), jnp.float32),
        ],
        scratch_shapes=[
            pltpu.VMEM((a_num, w_num), jnp.float32),
            pltpu.VMEM((e_num, t_num), jnp.float32),
        ],
    )(xtb, xb, at, ut, w2r, w1r)

    top = jnp.concatenate([at_new.T, padding_a], axis=1)
    bot = jnp.concatenate([ut_new.T, padding_t], axis=1)
    return jnp.concatenate([top, bot], axis=0)
